# baseline (speedup 1.0000x reference)
import functools
import math

import jax
import jax.numpy as jnp
from jax import lax
from jax.experimental import pallas as pl
from jax.experimental.pallas import tpu as pltpu
from jax.experimental.pallas import tpu_sc as plsc

NC = 2
NS = 16
NW = NC * NS
LANES = 16

GATHER_ROWS = 128
CHUNK = 256


def _make_pe(seq_len: int, d: int) -> jax.Array:
    pos = jnp.arange(0, seq_len, dtype=jnp.float32)[:, None]
    fill = pos * jnp.exp(
        -jnp.arange(0, d, 2, dtype=jnp.float32) * math.log(10000.0) / d
    )
    pe = jnp.zeros((seq_len, d), dtype=jnp.float32)
    pe = pe.at[:, 0::2].set(jnp.sin(fill))
    pe = pe.at[:, 1::2].set(jnp.cos(fill))
    return pe


@functools.partial(jax.jit, static_argnames=("n_rows", "d", "seq_len"))
def _sc_embed(x2d, pe, table2, *, n_rows, d, seq_len):
    per_w = n_rows // NW
    n_g = per_w // GATHER_ROWS
    n_chunks = per_w // CHUNK
    g_per_chunk = CHUNK // GATHER_ROWS
    vecs_per_row = d // LANES
    d2 = 2 * d

    mesh = plsc.VectorSubcoreMesh(core_axis_name="c", subcore_axis_name="s")

    @functools.partial(
        pl.kernel,
        out_type=jax.ShapeDtypeStruct((n_rows, d), jnp.float32),
        mesh=mesh,
        compiler_params=pltpu.CompilerParams(use_tc_tiling_on_sc=True),
        scratch_types=[
            pltpu.VMEM((n_g, GATHER_ROWS), jnp.int32),
            pltpu.VMEM((n_g, GATHER_ROWS), jnp.int32),
            pltpu.VMEM((per_w // LANES, LANES), jnp.float32),
            pltpu.VMEM((CHUNK, d2), jnp.float32),
            pltpu.VMEM((CHUNK, d), jnp.float32),
            pltpu.SemaphoreType.DMA,
        ],
    )
    def body(x_hbm, pe_hbm, table2_hbm, out_hbm,
             idx_v, pidx_v, par_v, pairs_v, out_v, sem):
        wid = lax.axis_index("s") * NC + lax.axis_index("c")
        base = wid * per_w
        l_start = lax.rem(base, seq_len)

        pltpu.sync_copy(x_hbm.at[pl.ds(wid * n_g, n_g)], idx_v)

        for v in range(per_w // LANES):
            r, c0 = v // (GATHER_ROWS // LANES), (v % (GATHER_ROWS // LANES)) * LANES
            tok = idx_v[r, pl.ds(c0, LANES)]
            pidx_v[r, pl.ds(c0, LANES)] = lax.shift_right_logical(tok, 1)
            par_v[v, :] = lax.convert_element_type(
                lax.bitwise_and(tok, 1), jnp.float32
            )
        for c in range(n_chunks):
            copies = [
                pltpu.async_copy(
                    table2_hbm.at[pidx_v.at[c * g_per_chunk + k]],
                    pairs_v.at[pl.ds(k * GATHER_ROWS, GATHER_ROWS)],
                    sem,
                )
                for k in range(g_per_chunk)
            ]
            pltpu.sync_copy(
                pe_hbm.at[pl.ds(l_start + c * CHUNK, CHUNK)], out_v
            )
            for cp in copies:
                cp.wait()

            def row_fix(i, carry):
                t = c * CHUNK + i
                g = lax.div(t, LANES)
                lane = lax.rem(t, LANES)
                par_vec = par_v[g, :]
                pf = par_vec[jnp.full((LANES,), lane, jnp.int32)]
                for j in range(vecs_per_row):
                    lo = pairs_v[i, pl.ds(j * LANES, LANES)]
                    hi = pairs_v[i, pl.ds(d + j * LANES, LANES)]
                    val = lo + pf * (hi - lo)
                    plsc.addupdate(out_v.at[i, pl.ds(j * LANES, LANES)], val)
                return carry

            lax.fori_loop(0, CHUNK, row_fix, 0)
            pltpu.sync_copy(out_v, out_hbm.at[pl.ds(base + c * CHUNK, CHUNK)])

    return body(x2d, pe, table2)


def kernel(x, table):
    b, l = x.shape
    v, d = table.shape
    n_rows = b * l
    pe = _make_pe(l, d)
    x2d = x.reshape(n_rows // GATHER_ROWS, GATHER_ROWS).astype(jnp.int32)
    table2 = table.reshape(v // 2, 2 * d)
    out = _sc_embed(x2d, pe, table2, n_rows=n_rows, d=d, seq_len=l)
    return out.reshape(b, l, d)
